# baseline (speedup 1.0000x reference)
import functools

import jax
import jax.numpy as jnp
from jax.experimental import pallas as pl
from jax.experimental.pallas import tpu as pltpu

_TOPK = 2
_BLK = 256
_IB = 512


def _ffn_body(bemap_ref, bevalid_ref, xs_ref, fc1_ref, fc3_ref, fc2_ref,
              out_ref, acc_ref, *, n_it):
    nb = pl.program_id(0)
    it = pl.program_id(1)

    @pl.when(bevalid_ref[nb] == 1)
    def _():
        @pl.when(it == 0)
        def _():
            acc_ref[...] = jnp.zeros_like(acc_ref)

        xs = xs_ref[...]
        h1 = jnp.dot(xs, fc1_ref[0], preferred_element_type=jnp.float32)
        h3 = jnp.dot(xs, fc3_ref[0], preferred_element_type=jnp.float32)
        act = h1 * jax.nn.sigmoid(h1) * h3
        acc_ref[...] += jnp.dot(act, fc2_ref[0], preferred_element_type=jnp.float32)

        @pl.when(it == n_it - 1)
        def _():
            out_ref[...] = acc_ref[...]


def _grouped_ffn(xs, fc1, fc3, fc2, bemap, bevalid):
    p, h = xs.shape
    _, _, inter = fc1.shape
    n_nb = p // _BLK
    n_it = inter // _IB
    return pl.pallas_call(
        functools.partial(_ffn_body, n_it=n_it),
        grid_spec=pltpu.PrefetchScalarGridSpec(
            num_scalar_prefetch=2,
            grid=(n_nb, n_it),
            in_specs=[
                pl.BlockSpec((_BLK, h), lambda nb, it, bm, bv: (nb, 0)),
                pl.BlockSpec((1, h, _IB), lambda nb, it, bm, bv: (bm[nb], 0, it)),
                pl.BlockSpec((1, h, _IB), lambda nb, it, bm, bv: (bm[nb], 0, it)),
                pl.BlockSpec((1, _IB, h), lambda nb, it, bm, bv: (bm[nb], it, 0)),
            ],
            out_specs=pl.BlockSpec((_BLK, h), lambda nb, it, bm, bv: (nb, 0)),
            scratch_shapes=[pltpu.VMEM((_BLK, h), jnp.float32)],
        ),
        out_shape=jax.ShapeDtypeStruct((p, h), jnp.float32),
        compiler_params=pltpu.CompilerParams(
            dimension_semantics=("arbitrary", "arbitrary"),
        ),
    )(bemap, bevalid, xs, fc1, fc3, fc2)


def kernel(x, router_probs, fc1, fc2, fc3):
    n, h = x.shape
    e = fc1.shape[0]
    nk = n * _TOPK

    topk_probs, topk_idx = jax.lax.top_k(router_probs, _TOPK)
    topk_probs = topk_probs / jnp.sum(topk_probs, axis=1, keepdims=True)
    flat_e = topk_idx.reshape(-1).astype(jnp.int32)

    order = jnp.argsort(flat_e)
    sorted_e = flat_e[order]
    counts = jnp.zeros((e,), jnp.int32).at[flat_e].add(1)
    padded = ((counts + _BLK - 1) // _BLK) * _BLK
    pstart = jnp.concatenate(
        [jnp.zeros((1,), jnp.int32), jnp.cumsum(padded)[:-1].astype(jnp.int32)])
    gstart = jnp.concatenate(
        [jnp.zeros((1,), jnp.int32), jnp.cumsum(counts)[:-1].astype(jnp.int32)])
    ranks = jnp.arange(nk, dtype=jnp.int32) - gstart[sorted_e]
    pos_sorted = pstart[sorted_e] + ranks
    pos = jnp.zeros((nk,), jnp.int32).at[order].set(pos_sorted)

    ptotal = nk + e * _BLK
    n_nb = ptotal // _BLK
    slot_token = jnp.zeros((ptotal,), jnp.int32).at[pos_sorted].set(
        (order // _TOPK).astype(jnp.int32))

    ends = jnp.cumsum(padded).astype(jnp.int32)
    blk_starts = jnp.arange(n_nb, dtype=jnp.int32) * _BLK
    be = jnp.sum(blk_starts[:, None] >= ends[None, :], axis=1).astype(jnp.int32)
    bevalid = (be < e).astype(jnp.int32)
    bemap = jnp.minimum(be, e - 1)

    xs = jnp.take(x, slot_token, axis=0)
    ys = _grouped_ffn(xs, fc1, fc3, fc2, bemap, bevalid)
    gath = jnp.take(ys, pos, axis=0).reshape(n, _TOPK, h)
    return jnp.sum(gath * topk_probs[:, :, None], axis=1)


# speedup vs baseline: 1.3090x; 1.3090x over previous
import functools

import jax
import jax.numpy as jnp
from jax.experimental import pallas as pl
from jax.experimental.pallas import tpu as pltpu

_TOPK = 2
_BLK = 576
_IB = 1024


def _ffn_body(bemap_ref, nbc_ref, valid_ref, xs_ref, fc1_ref, fc3_ref,
              fc2_ref, out_ref, acc_ref, *, n_it):
    nb = pl.program_id(0)
    it = pl.program_id(1)

    @pl.when(valid_ref[nb] == 1)
    def _():
        @pl.when(it == 0)
        def _():
            acc_ref[...] = jnp.zeros_like(acc_ref)

        xs = xs_ref[...]
        h1 = jnp.dot(xs, fc1_ref[0], preferred_element_type=jnp.float32)
        h3 = jnp.dot(xs, fc3_ref[0], preferred_element_type=jnp.float32)
        act = h1 * jax.nn.sigmoid(h1) * h3
        acc_ref[...] += jnp.dot(act, fc2_ref[0], preferred_element_type=jnp.float32)

        @pl.when(it == n_it - 1)
        def _():
            out_ref[...] = acc_ref[...]


def _grouped_ffn(xs, fc1, fc3, fc2, bemap, nbclamp, valid):
    p, h = xs.shape
    _, _, inter = fc1.shape
    n_nb = p // _BLK
    n_it = inter // _IB

    def wmap(nb, it, bm, nc, vl):
        return (bm[nb], jnp.where(vl[nb] == 1, it, n_it - 1))

    def map13(nb, it, bm, nc, vl):
        be_i, it_i = wmap(nb, it, bm, nc, vl)
        return (be_i, 0, it_i)

    def map2(nb, it, bm, nc, vl):
        be_i, it_i = wmap(nb, it, bm, nc, vl)
        return (be_i, it_i, 0)

    return pl.pallas_call(
        functools.partial(_ffn_body, n_it=n_it),
        grid_spec=pltpu.PrefetchScalarGridSpec(
            num_scalar_prefetch=3,
            grid=(n_nb, n_it),
            in_specs=[
                pl.BlockSpec((_BLK, h), lambda nb, it, bm, nc, vl: (nc[nb], 0)),
                pl.BlockSpec((1, h, _IB), map13),
                pl.BlockSpec((1, h, _IB), map13),
                pl.BlockSpec((1, _IB, h), map2),
            ],
            out_specs=pl.BlockSpec((_BLK, h), lambda nb, it, bm, nc, vl: (nc[nb], 0)),
            scratch_shapes=[pltpu.VMEM((_BLK, h), jnp.float32)],
        ),
        out_shape=jax.ShapeDtypeStruct((p, h), jnp.float32),
        compiler_params=pltpu.CompilerParams(
            dimension_semantics=("arbitrary", "arbitrary"),
        ),
    )(bemap, nbclamp, valid, xs, fc1, fc3, fc2)


def kernel(x, router_probs, fc1, fc2, fc3):
    n, h = x.shape
    e = fc1.shape[0]
    nk = n * _TOPK

    topk_probs, topk_idx = jax.lax.top_k(router_probs, _TOPK)
    topk_probs = topk_probs / jnp.sum(topk_probs, axis=1, keepdims=True)
    flat_e = topk_idx.reshape(-1).astype(jnp.int32)

    order = jnp.argsort(flat_e)
    sorted_e = flat_e[order]
    counts = jnp.zeros((e,), jnp.int32).at[flat_e].add(1)
    padded = ((counts + _BLK - 1) // _BLK) * _BLK
    pstart = jnp.concatenate(
        [jnp.zeros((1,), jnp.int32), jnp.cumsum(padded)[:-1].astype(jnp.int32)])
    gstart = jnp.concatenate(
        [jnp.zeros((1,), jnp.int32), jnp.cumsum(counts)[:-1].astype(jnp.int32)])
    ranks = jnp.arange(nk, dtype=jnp.int32) - gstart[sorted_e]
    pos_sorted = pstart[sorted_e] + ranks
    pos = jnp.zeros((nk,), jnp.int32).at[order].set(pos_sorted)

    ptotal = -(-(nk + e * _BLK) // _BLK) * _BLK
    n_nb = ptotal // _BLK
    slot_token = jnp.zeros((ptotal,), jnp.int32).at[pos_sorted].set(
        (order // _TOPK).astype(jnp.int32))

    ends = jnp.cumsum(padded).astype(jnp.int32)
    blk_starts = jnp.arange(n_nb, dtype=jnp.int32) * _BLK
    be = jnp.sum(blk_starts[:, None] >= ends[None, :], axis=1).astype(jnp.int32)
    valid = (be < e).astype(jnp.int32)
    last_valid = jnp.maximum(jnp.sum(valid) - 1, 0)
    nbclamp = jnp.minimum(jnp.arange(n_nb, dtype=jnp.int32), last_valid)
    bemap = jnp.minimum(be, be[last_valid])

    xs = jnp.take(x, slot_token, axis=0)
    ys = _grouped_ffn(xs, fc1, fc3, fc2, bemap, nbclamp, valid)
    gath = jnp.take(ys, pos, axis=0).reshape(n, _TOPK, h)
    return jnp.sum(gath * topk_probs[:, :, None], axis=1)


# speedup vs baseline: 1.6095x; 1.2296x over previous
import functools

import jax
import jax.numpy as jnp
from jax.experimental import pallas as pl
from jax.experimental.pallas import tpu as pltpu

_TOPK = 2
_BLK = 576
_IB = 1024


def _ffn_body(bemap_ref, nbc_ref, valid_ref, xs_ref, fc1_ref, fc3_ref,
              fc2_ref, out_ref, acc_ref, *, n_it):
    nb = pl.program_id(0)
    it = pl.program_id(1)

    @pl.when(valid_ref[nb] == 1)
    def _():
        @pl.when(it == 0)
        def _():
            acc_ref[...] = jnp.zeros_like(acc_ref)

        xs = xs_ref[...]
        h1 = jnp.dot(xs, fc1_ref[0], preferred_element_type=jnp.float32)
        h3 = jnp.dot(xs, fc3_ref[0], preferred_element_type=jnp.float32)
        act = h1 * jax.nn.sigmoid(h1) * h3
        acc_ref[...] += jnp.dot(act, fc2_ref[0], preferred_element_type=jnp.float32)

        @pl.when(it == n_it - 1)
        def _():
            out_ref[...] = acc_ref[...]


def _grouped_ffn(xs, fc1, fc3, fc2, bemap, nbclamp, valid):
    p, h = xs.shape
    _, _, inter = fc1.shape
    n_nb = p // _BLK
    n_it = inter // _IB

    def wmap(nb, it, bm, nc, vl):
        return (bm[nb], jnp.where(vl[nb] == 1, it, n_it - 1))

    def map13(nb, it, bm, nc, vl):
        be_i, it_i = wmap(nb, it, bm, nc, vl)
        return (be_i, 0, it_i)

    def map2(nb, it, bm, nc, vl):
        be_i, it_i = wmap(nb, it, bm, nc, vl)
        return (be_i, it_i, 0)

    return pl.pallas_call(
        functools.partial(_ffn_body, n_it=n_it),
        grid_spec=pltpu.PrefetchScalarGridSpec(
            num_scalar_prefetch=3,
            grid=(n_nb, n_it),
            in_specs=[
                pl.BlockSpec((_BLK, h), lambda nb, it, bm, nc, vl: (nc[nb], 0)),
                pl.BlockSpec((1, h, _IB), map13),
                pl.BlockSpec((1, h, _IB), map13),
                pl.BlockSpec((1, _IB, h), map2),
            ],
            out_specs=pl.BlockSpec((_BLK, h), lambda nb, it, bm, nc, vl: (nc[nb], 0)),
            scratch_shapes=[pltpu.VMEM((_BLK, h), jnp.float32)],
        ),
        out_shape=jax.ShapeDtypeStruct((p, h), jnp.float32),
        compiler_params=pltpu.CompilerParams(
            dimension_semantics=("arbitrary", "arbitrary"),
        ),
    )(bemap, nbclamp, valid, xs, fc1, fc3, fc2)


def kernel(x, router_probs, fc1, fc2, fc3):
    n, h = x.shape
    e = fc1.shape[0]
    nk = n * _TOPK

    topk_probs, topk_idx = jax.lax.top_k(router_probs, _TOPK)
    topk_probs = topk_probs / jnp.sum(topk_probs, axis=1, keepdims=True)
    flat_e = topk_idx.reshape(-1).astype(jnp.int32)

    oh = (flat_e[:, None] == jnp.arange(e, dtype=jnp.int32)[None, :]).astype(jnp.int32)
    csum = jnp.cumsum(oh, axis=0)
    counts = csum[-1]
    rank = jnp.sum(csum * oh, axis=1) - 1

    padded = ((counts + _BLK - 1) // _BLK) * _BLK
    pstart = jnp.concatenate(
        [jnp.zeros((1,), jnp.int32), jnp.cumsum(padded)[:-1].astype(jnp.int32)])
    pos = pstart[flat_e] + rank

    ptotal = -(-(nk + e * _BLK) // _BLK) * _BLK
    n_nb = ptotal // _BLK

    ends = jnp.cumsum(padded).astype(jnp.int32)
    blk_starts = jnp.arange(n_nb, dtype=jnp.int32) * _BLK
    be = jnp.sum(blk_starts[:, None] >= ends[None, :], axis=1).astype(jnp.int32)
    valid = (be < e).astype(jnp.int32)
    last_valid = jnp.maximum(jnp.sum(valid) - 1, 0)
    nbclamp = jnp.minimum(jnp.arange(n_nb, dtype=jnp.int32), last_valid)
    bemap = jnp.minimum(be, be[last_valid])

    x2 = jnp.broadcast_to(x[:, None, :], (n, _TOPK, h)).reshape(nk, h)
    xs = jnp.zeros((ptotal, h), x.dtype).at[pos].set(x2)
    ys = _grouped_ffn(xs, fc1, fc3, fc2, bemap, nbclamp, valid)
    gath = jnp.take(ys, pos, axis=0).reshape(n, _TOPK, h)
    return jnp.sum(gath * topk_probs[:, :, None], axis=1)


# speedup vs baseline: 1.8331x; 1.1389x over previous
import functools

import jax
import jax.numpy as jnp
from jax import lax
from jax.experimental import pallas as pl
from jax.experimental.pallas import tpu as pltpu
from jax.experimental.pallas import tpu_sc as plsc

_NC, _NS, _NL = 2, 16, 16

_TOPK = 2
_BLK = 576
_IB = 1024


def _ffn_body(bemap_ref, nbc_ref, valid_ref, xs_ref, fc1_ref, fc3_ref,
              fc2_ref, out_ref, acc_ref, *, n_it):
    nb = pl.program_id(0)
    it = pl.program_id(1)

    @pl.when(valid_ref[nb] == 1)
    def _():
        @pl.when(it == 0)
        def _():
            acc_ref[...] = jnp.zeros_like(acc_ref)

        xs = xs_ref[...]
        h1 = jnp.dot(xs, fc1_ref[0], preferred_element_type=jnp.float32)
        h3 = jnp.dot(xs, fc3_ref[0], preferred_element_type=jnp.float32)
        act = h1 * jax.nn.sigmoid(h1) * h3
        acc_ref[...] += jnp.dot(act, fc2_ref[0], preferred_element_type=jnp.float32)

        @pl.when(it == n_it - 1)
        def _():
            out_ref[...] = acc_ref[...]


def _grouped_ffn(xs, fc1, fc3, fc2, bemap, nbclamp, valid):
    p, h = xs.shape
    _, _, inter = fc1.shape
    n_nb = p // _BLK
    n_it = inter // _IB

    def wmap(nb, it, bm, nc, vl):
        return (bm[nb], jnp.where(vl[nb] == 1, it, n_it - 1))

    def map13(nb, it, bm, nc, vl):
        be_i, it_i = wmap(nb, it, bm, nc, vl)
        return (be_i, 0, it_i)

    def map2(nb, it, bm, nc, vl):
        be_i, it_i = wmap(nb, it, bm, nc, vl)
        return (be_i, it_i, 0)

    return pl.pallas_call(
        functools.partial(_ffn_body, n_it=n_it),
        grid_spec=pltpu.PrefetchScalarGridSpec(
            num_scalar_prefetch=3,
            grid=(n_nb, n_it),
            in_specs=[
                pl.BlockSpec((_BLK, h), lambda nb, it, bm, nc, vl: (nc[nb], 0)),
                pl.BlockSpec((1, h, _IB), map13),
                pl.BlockSpec((1, h, _IB), map13),
                pl.BlockSpec((1, _IB, h), map2),
            ],
            out_specs=pl.BlockSpec((_BLK, h), lambda nb, it, bm, nc, vl: (nc[nb], 0)),
            scratch_shapes=[pltpu.VMEM((_BLK, h), jnp.float32)],
        ),
        out_shape=jax.ShapeDtypeStruct((p, h), jnp.float32),
        compiler_params=pltpu.CompilerParams(
            dimension_semantics=("arbitrary", "arbitrary"),
        ),
    )(bemap, nbclamp, valid, xs, fc1, fc3, fc2)


def _vgather(vec, idx):
    return lax.gather(
        vec, idx[:, None],
        lax.GatherDimensionNumbers(offset_dims=(), collapsed_slice_dims=(0,),
                                   start_index_map=(0,)),
        (1,), mode=lax.GatherScatterMode.PROMISE_IN_BOUNDS)


def _vsplat(vec, i):
    return _vgather(vec, jnp.full((_NL,), i, dtype=jnp.int32))


def _combine_sc(ys, posr, wr, n, h):
    nw = _NC * _NS
    tpw = n // nw
    nch = tpw // _NL
    mesh = plsc.VectorSubcoreMesh(core_axis_name="c", subcore_axis_name="s")

    @functools.partial(
        pl.kernel, mesh=mesh,
        out_type=jax.ShapeDtypeStruct((n, h), jnp.float32),
        scratch_types=[
            pltpu.VMEM((nch, 2 * _NL), jnp.int32),
            pltpu.VMEM((nch * 2 * _NL + _NL,), jnp.float32),
            pltpu.VMEM((2 * _NL, h), jnp.float32),
            pltpu.VMEM((_NL, h), jnp.float32),
            pltpu.SemaphoreType.DMA,
        ],
    )
    def body(ys_hbm, pos_hbm, w_hbm, out_hbm, idx_v, w_v, rows_v, out_v, sem):
        wid = lax.axis_index("s") * _NC + lax.axis_index("c")
        pltpu.sync_copy(pos_hbm.at[pl.ds(wid * nch, nch)], idx_v)
        pltpu.sync_copy(w_hbm.at[pl.ds(wid * nch * 2 * _NL, nch * 2 * _NL)],
                        w_v.at[pl.ds(0, nch * 2 * _NL)])
        for c in range(nch):
            pltpu.async_copy(ys_hbm.at[idx_v.at[c]], rows_v, sem).wait()

            def tok(t, _):
                wpair = w_v[pl.ds(c * 2 * _NL + 2 * t, _NL)]
                w1s = jnp.full((_NL,), wpair[0])
                w2s = jnp.full((_NL,), wpair[1])
                for sl in range(h // _NL):
                    d = pl.ds(sl * _NL, _NL)
                    out_v[t, d] = (w1s * rows_v[2 * t, d]
                                   + w2s * rows_v[2 * t + 1, d])
                return 0

            lax.fori_loop(0, _NL, tok, 0)
            pltpu.sync_copy(out_v, out_hbm.at[pl.ds(wid * tpw + c * _NL, _NL)])

    return body(ys, posr, wr)


def kernel(x, router_probs, fc1, fc2, fc3):
    n, h = x.shape
    e = fc1.shape[0]
    nk = n * _TOPK

    topk_probs, topk_idx = jax.lax.top_k(router_probs, _TOPK)
    topk_probs = topk_probs / jnp.sum(topk_probs, axis=1, keepdims=True)
    flat_e = topk_idx.reshape(-1).astype(jnp.int32)

    oh = (flat_e[:, None] == jnp.arange(e, dtype=jnp.int32)[None, :]).astype(jnp.int32)
    csum = jnp.cumsum(oh, axis=0)
    counts = csum[-1]
    rank = jnp.sum(csum * oh, axis=1) - 1

    padded = ((counts + _BLK - 1) // _BLK) * _BLK
    pstart = jnp.concatenate(
        [jnp.zeros((1,), jnp.int32), jnp.cumsum(padded)[:-1].astype(jnp.int32)])
    pos = pstart[flat_e] + rank

    ptotal = -(-(nk + e * _BLK) // _BLK) * _BLK
    n_nb = ptotal // _BLK

    ends = jnp.cumsum(padded).astype(jnp.int32)
    blk_starts = jnp.arange(n_nb, dtype=jnp.int32) * _BLK
    be = jnp.sum(blk_starts[:, None] >= ends[None, :], axis=1).astype(jnp.int32)
    valid = (be < e).astype(jnp.int32)
    last_valid = jnp.maximum(jnp.sum(valid) - 1, 0)
    nbclamp = jnp.minimum(jnp.arange(n_nb, dtype=jnp.int32), last_valid)
    bemap = jnp.minimum(be, be[last_valid])

    x2 = jnp.broadcast_to(x[:, None, :], (n, _TOPK, h)).reshape(nk, h)
    xs = jnp.zeros((ptotal, h), x.dtype).at[pos].set(x2)
    ys = _grouped_ffn(xs, fc1, fc3, fc2, bemap, nbclamp, valid)
    posr = pos.reshape(nk // (2 * _NL), 2 * _NL)
    wflat = topk_probs.reshape(-1)
    return _combine_sc(ys, posr, wflat, n, h)


# speedup vs baseline: 2.4301x; 1.3257x over previous
import functools

import jax
import jax.numpy as jnp
from jax import lax
from jax.experimental import pallas as pl
from jax.experimental.pallas import tpu as pltpu
from jax.experimental.pallas import tpu_sc as plsc

_NC, _NS, _NL = 2, 16, 16

_TOPK = 2
_BLK = 576
_IB = 1024


def _ffn_body(bemap_ref, nbc_ref, valid_ref, xs_ref, fc1_ref, fc3_ref,
              fc2_ref, out_ref, acc_ref, *, n_it):
    nb = pl.program_id(0)
    it = pl.program_id(1)

    @pl.when(valid_ref[nb] == 1)
    def _():
        @pl.when(it == 0)
        def _():
            acc_ref[...] = jnp.zeros_like(acc_ref)

        xs = xs_ref[...]
        h1 = jnp.dot(xs, fc1_ref[0], preferred_element_type=jnp.float32)
        h3 = jnp.dot(xs, fc3_ref[0], preferred_element_type=jnp.float32)
        act = h1 * jax.nn.sigmoid(h1) * h3
        acc_ref[...] += jnp.dot(act, fc2_ref[0], preferred_element_type=jnp.float32)

        @pl.when(it == n_it - 1)
        def _():
            out_ref[...] = acc_ref[...]


def _grouped_ffn(xs, fc1, fc3, fc2, bemap, nbclamp, valid):
    p, h = xs.shape
    _, _, inter = fc1.shape
    n_nb = p // _BLK
    n_it = inter // _IB

    def wmap(nb, it, bm, nc, vl):
        return (bm[nb], jnp.where(vl[nb] == 1, it, n_it - 1))

    def map13(nb, it, bm, nc, vl):
        be_i, it_i = wmap(nb, it, bm, nc, vl)
        return (be_i, 0, it_i)

    def map2(nb, it, bm, nc, vl):
        be_i, it_i = wmap(nb, it, bm, nc, vl)
        return (be_i, it_i, 0)

    return pl.pallas_call(
        functools.partial(_ffn_body, n_it=n_it),
        grid_spec=pltpu.PrefetchScalarGridSpec(
            num_scalar_prefetch=3,
            grid=(n_nb, n_it),
            in_specs=[
                pl.BlockSpec((_BLK, h), lambda nb, it, bm, nc, vl: (nc[nb], 0)),
                pl.BlockSpec((1, h, _IB), map13),
                pl.BlockSpec((1, h, _IB), map13),
                pl.BlockSpec((1, _IB, h), map2),
            ],
            out_specs=pl.BlockSpec((_BLK, h), lambda nb, it, bm, nc, vl: (nc[nb], 0)),
            scratch_shapes=[pltpu.VMEM((_BLK, h), jnp.float32)],
        ),
        out_shape=jax.ShapeDtypeStruct((p, h), jnp.float32),
        compiler_params=pltpu.CompilerParams(
            dimension_semantics=("arbitrary", "arbitrary"),
        ),
    )(bemap, nbclamp, valid, xs, fc1, fc3, fc2)


def _vgather(vec, idx):
    return lax.gather(
        vec, idx[:, None],
        lax.GatherDimensionNumbers(offset_dims=(), collapsed_slice_dims=(0,),
                                   start_index_map=(0,)),
        (1,), mode=lax.GatherScatterMode.PROMISE_IN_BOUNDS)


def _vsplat(vec, i):
    return _vgather(vec, jnp.full((_NL,), i, dtype=jnp.int32))


def _scatter_sc(x, pos1, pos2, n, h, ptotal):
    nw = _NC * _NS
    tpw = n // nw
    mesh = plsc.VectorSubcoreMesh(core_axis_name="c", subcore_axis_name="s")

    @functools.partial(
        pl.kernel, mesh=mesh,
        out_type=jax.ShapeDtypeStruct((ptotal, h), jnp.float32),
        scratch_types=[
            pltpu.VMEM((tpw, h), jnp.float32),
            pltpu.VMEM((2, tpw), jnp.int32),
            pltpu.SemaphoreType.DMA,
        ],
    )
    def body(x_hbm, pos1_hbm, pos2_hbm, xs_hbm, xrows_v, idx2_v, sem):
        wid = lax.axis_index("s") * _NC + lax.axis_index("c")
        base = pl.ds(wid * tpw, tpw)
        pltpu.sync_copy(pos1_hbm.at[base], idx2_v.at[0])
        pltpu.sync_copy(pos2_hbm.at[base], idx2_v.at[1])
        pltpu.sync_copy(x_hbm.at[base], xrows_v)
        pltpu.async_copy(xrows_v, xs_hbm.at[idx2_v.at[0]], sem).wait()
        pltpu.async_copy(xrows_v, xs_hbm.at[idx2_v.at[1]], sem).wait()

    return body(x, pos1, pos2)


def _combine_sc(ys, posr, wr, n, h):
    nw = _NC * _NS
    tpw = n // nw
    nch = tpw // _NL
    mesh = plsc.VectorSubcoreMesh(core_axis_name="c", subcore_axis_name="s")

    @functools.partial(
        pl.kernel, mesh=mesh,
        out_type=jax.ShapeDtypeStruct((n, h), jnp.float32),
        scratch_types=[
            pltpu.VMEM((nch, 2 * _NL), jnp.int32),
            pltpu.VMEM((nch * 2 * _NL + _NL,), jnp.float32),
            pltpu.VMEM((2 * _NL, h), jnp.float32),
            pltpu.VMEM((_NL, h), jnp.float32),
            pltpu.SemaphoreType.DMA,
        ],
    )
    def body(ys_hbm, pos_hbm, w_hbm, out_hbm, idx_v, w_v, rows_v, out_v, sem):
        wid = lax.axis_index("s") * _NC + lax.axis_index("c")
        pltpu.sync_copy(pos_hbm.at[pl.ds(wid * nch, nch)], idx_v)
        pltpu.sync_copy(w_hbm.at[pl.ds(wid * nch * 2 * _NL, nch * 2 * _NL)],
                        w_v.at[pl.ds(0, nch * 2 * _NL)])
        for c in range(nch):
            pltpu.async_copy(ys_hbm.at[idx_v.at[c]], rows_v, sem).wait()

            def tok(t, _):
                wpair = w_v[pl.ds(c * 2 * _NL + 2 * t, _NL)]
                w1s = jnp.full((_NL,), wpair[0])
                w2s = jnp.full((_NL,), wpair[1])
                for sl in range(h // _NL):
                    d = pl.ds(sl * _NL, _NL)
                    out_v[t, d] = (w1s * rows_v[2 * t, d]
                                   + w2s * rows_v[2 * t + 1, d])
                return 0

            lax.fori_loop(0, _NL, tok, 0)
            pltpu.sync_copy(out_v, out_hbm.at[pl.ds(wid * tpw + c * _NL, _NL)])

    return body(ys, posr, wr)


def kernel(x, router_probs, fc1, fc2, fc3):
    n, h = x.shape
    e = fc1.shape[0]
    nk = n * _TOPK

    topk_probs, topk_idx = jax.lax.top_k(router_probs, _TOPK)
    topk_probs = topk_probs / jnp.sum(topk_probs, axis=1, keepdims=True)
    flat_e = topk_idx.reshape(-1).astype(jnp.int32)

    oh = (flat_e[:, None] == jnp.arange(e, dtype=jnp.int32)[None, :]).astype(jnp.int32)
    csum = jnp.cumsum(oh, axis=0)
    counts = csum[-1]
    rank = jnp.sum(csum * oh, axis=1) - 1

    padded = ((counts + _BLK - 1) // _BLK) * _BLK
    pstart = jnp.concatenate(
        [jnp.zeros((1,), jnp.int32), jnp.cumsum(padded)[:-1].astype(jnp.int32)])
    pos = pstart[flat_e] + rank

    ptotal = -(-(nk + e * _BLK) // _BLK) * _BLK
    n_nb = ptotal // _BLK

    ends = jnp.cumsum(padded).astype(jnp.int32)
    blk_starts = jnp.arange(n_nb, dtype=jnp.int32) * _BLK
    be = jnp.sum(blk_starts[:, None] >= ends[None, :], axis=1).astype(jnp.int32)
    valid = (be < e).astype(jnp.int32)
    last_valid = jnp.maximum(jnp.sum(valid) - 1, 0)
    nbclamp = jnp.minimum(jnp.arange(n_nb, dtype=jnp.int32), last_valid)
    bemap = jnp.minimum(be, be[last_valid])

    pos2d = pos.reshape(n, _TOPK)
    xs = _scatter_sc(x, pos2d[:, 0], pos2d[:, 1], n, h, ptotal)
    ys = _grouped_ffn(xs, fc1, fc3, fc2, bemap, nbclamp, valid)
    posr = pos.reshape(nk // (2 * _NL), 2 * _NL)
    wflat = topk_probs.reshape(-1)
    return _combine_sc(ys, posr, wflat, n, h)
